# Initial kernel scaffold; baseline (speedup 1.0000x reference)
import jax
import jax.numpy as jnp
from jax.experimental import pallas as pl


def kernel(inputs, seq_len, params):
    raise NotImplementedError("write your pallas kernel here")


# speedup vs baseline: 3.0125x; 3.0125x over previous
import functools
import math

import jax
import jax.numpy as jnp
import numpy as np
from jax.experimental import pallas as pl
from jax.experimental.pallas import tpu as pltpu

B, T, IN_DIM = 4, 512, 80
D_MODEL, HIDDEN, OUT_DIM = 512, 1024, 2048
N_HEAD, N_MEM = 8, 64
LB, LA, SL, SR = 4, 1, 2, 1
N_EXPERTS = 4

_F32 = jnp.float32


def _pe_const():
    position = np.arange(T)[:, None].astype(np.float32)
    div_term = np.exp(np.arange(0, D_MODEL, 2).astype(np.float32)
                      * -(math.log(10000.0) / D_MODEL))
    pe = np.zeros((T, D_MODEL), dtype=np.float32)
    pe[:, 0::2] = np.sin(position * div_term)
    pe[:, 1::2] = np.cos(position * div_term)
    return pe[None]


_PE = _pe_const()


def _memory_block(vv, A, C):
    m = vv
    for i in range(LB):
        s = (i + 1) * SL
        shifted = jnp.concatenate(
            [jnp.zeros((s, vv.shape[1]), vv.dtype), vv[:T - s]], axis=0)
        m = m + shifted * A[i:i + 1, :]
    for j in range(LA):
        s = (j + 1) * SR
        shifted = jnp.concatenate(
            [vv[s:], jnp.zeros((s, vv.shape[1]), vv.dtype)], axis=0)
        m = m + shifted * C[j:j + 1, :]
    return m


def _dot(a, b):
    return jax.lax.dot(a, b, preferred_element_type=_F32)


def _fsmn_plain_kernel(x_ref, P_ref, bp_ref, V_ref, bv_ref, A_ref, C_ref,
                       o_ref, *, skip):
    x = x_ref[0]
    h = jnp.maximum(_dot(x, P_ref[...]) + bp_ref[...], 0.0)
    vv = _dot(h, V_ref[...]) + bv_ref[...]
    m = _memory_block(vv, A_ref[...], C_ref[...])
    if skip:
        m = m + x
    o_ref[0] = m


def _fsmn_plain(x, lp, skip):
    in_d = x.shape[-1]
    return pl.pallas_call(
        functools.partial(_fsmn_plain_kernel, skip=skip),
        grid=(B,),
        in_specs=[
            pl.BlockSpec((1, T, in_d), lambda b: (b, 0, 0)),
            pl.BlockSpec((in_d, HIDDEN), lambda b: (0, 0)),
            pl.BlockSpec((1, HIDDEN), lambda b: (0, 0)),
            pl.BlockSpec((HIDDEN, D_MODEL), lambda b: (0, 0)),
            pl.BlockSpec((1, D_MODEL), lambda b: (0, 0)),
            pl.BlockSpec((LB, D_MODEL), lambda b: (0, 0)),
            pl.BlockSpec((LA, D_MODEL), lambda b: (0, 0)),
        ],
        out_specs=pl.BlockSpec((1, T, D_MODEL), lambda b: (b, 0, 0)),
        out_shape=jax.ShapeDtypeStruct((B, T, D_MODEL), _F32),
    )(x, lp["P"], lp["bp"].reshape(1, -1), lp["V"], lp["bv"].reshape(1, -1),
      lp["A"], lp["C"])


def _san_kernel(x_ref, mb_ref, Wq_ref, bq_ref, Wk_ref, bk_ref, Wv_ref, bv_ref,
                Wo_ref, bo_ref, MK_ref, MV_ref, g_ref, beta_ref, *rest,
                add_pe):
    if add_pe:
        pe_ref, o_ref = rest
    else:
        (o_ref,) = rest
    x = x_ref[0]
    if add_pe:
        x = x + pe_ref[0]
    q = _dot(x, Wq_ref[...]) + bq_ref[...]
    k = _dot(x, Wk_ref[...]) + bk_ref[...]
    v = _dot(x, Wv_ref[...]) + bv_ref[...]
    K = jnp.concatenate([k, MK_ref[...]], axis=0)
    V = jnp.concatenate([v, MV_ref[...]], axis=0)
    mb = mb_ref[0]
    dh = D_MODEL // N_HEAD
    scale = 1.0 / math.sqrt(dh)
    outs = []
    for hh in range(N_HEAD):
        sl = slice(hh * dh, (hh + 1) * dh)
        qh = q[:, sl]
        kh = K[:, sl]
        vh = V[:, sl]
        s = jax.lax.dot_general(qh, kh, (((1,), (1,)), ((), ())),
                                preferred_element_type=_F32) * scale
        s = s + mb
        mx = jnp.max(s, axis=-1, keepdims=True)
        ee = jnp.exp(s - mx)
        attn = ee / jnp.sum(ee, axis=-1, keepdims=True)
        outs.append(_dot(attn, vh))
    o = jnp.concatenate(outs, axis=1)
    o = _dot(o, Wo_ref[...]) + bo_ref[...]
    y = x + o
    mu = jnp.mean(y, axis=-1, keepdims=True)
    var = jnp.mean((y - mu) ** 2, axis=-1, keepdims=True)
    o_ref[0] = (y - mu) * jax.lax.rsqrt(var + 1e-5) * g_ref[...] + beta_ref[...]


def _san(x, maskb, p, pe=None):
    add_pe = pe is not None
    specs = [
        pl.BlockSpec((1, T, D_MODEL), lambda b: (b, 0, 0)),
        pl.BlockSpec((1, 1, T + N_MEM), lambda b: (b, 0, 0)),
        pl.BlockSpec((D_MODEL, D_MODEL), lambda b: (0, 0)),
        pl.BlockSpec((1, D_MODEL), lambda b: (0, 0)),
        pl.BlockSpec((D_MODEL, D_MODEL), lambda b: (0, 0)),
        pl.BlockSpec((1, D_MODEL), lambda b: (0, 0)),
        pl.BlockSpec((D_MODEL, D_MODEL), lambda b: (0, 0)),
        pl.BlockSpec((1, D_MODEL), lambda b: (0, 0)),
        pl.BlockSpec((D_MODEL, D_MODEL), lambda b: (0, 0)),
        pl.BlockSpec((1, D_MODEL), lambda b: (0, 0)),
        pl.BlockSpec((N_MEM, D_MODEL), lambda b: (0, 0)),
        pl.BlockSpec((N_MEM, D_MODEL), lambda b: (0, 0)),
        pl.BlockSpec((1, D_MODEL), lambda b: (0, 0)),
        pl.BlockSpec((1, D_MODEL), lambda b: (0, 0)),
    ]
    args = [x, maskb, p["Wq"], p["bq"].reshape(1, -1), p["Wk"],
            p["bk"].reshape(1, -1), p["Wv"], p["bv"].reshape(1, -1),
            p["Wo"], p["bo"].reshape(1, -1), p["MemK"], p["MemV"],
            p["g"].reshape(1, -1), p["beta"].reshape(1, -1)]
    if add_pe:
        specs.append(pl.BlockSpec((1, T, D_MODEL), lambda b: (0, 0, 0)))
        args.append(pe)
    return pl.pallas_call(
        functools.partial(_san_kernel, add_pe=add_pe),
        grid=(B,),
        in_specs=specs,
        out_specs=pl.BlockSpec((1, T, D_MODEL), lambda b: (b, 0, 0)),
        out_shape=jax.ShapeDtypeStruct((B, T, D_MODEL), _F32),
    )(*args)


def _moe_kernel(x_ref, emb_ref, Wr_ref, P_ref, bp_ref, V_ref, bv_ref, A_ref,
                C_ref, o_ref, acc_ref, aux_ref, *, skip):
    b = pl.program_id(0)
    x = x_ref[0]
    emb = emb_ref[0]
    logits = _dot(emb, Wr_ref[...])
    mxl = jnp.max(logits, axis=-1, keepdims=True)
    ex = jnp.exp(logits - mxl)
    gates = ex / jnp.sum(ex, axis=-1, keepdims=True)
    mxg = jnp.max(gates, axis=-1, keepdims=True)

    run = jnp.zeros((T, 1), _F32)
    oh = []
    for e in range(N_EXPERTS):
        eq = (gates[:, e:e + 1] >= mxg).astype(_F32)
        oh.append(eq * jnp.where(run < 0.5, 1.0, 0.0) * mxg)
        run = run + eq

    bp = bp_ref[...]
    bv = bv_ref[...]
    vv = jnp.zeros((T, D_MODEL), _F32)
    for e in range(N_EXPERTS):
        h = jnp.maximum(_dot(x, P_ref[e]) + bp[e:e + 1, :], 0.0)
        ve = _dot(h, V_ref[e]) + bv[e:e + 1, :]
        vv = vv + ve * oh[e]
    m = _memory_block(vv, A_ref[...], C_ref[...])
    if skip:
        m = m + x
    o_ref[0] = m

    gsum = jnp.sum(gates, axis=0, keepdims=True)
    prev = jnp.where(b == 0, jnp.zeros_like(gsum), acc_ref[...])
    acc_ref[...] = prev + gsum

    @pl.when(b == B - 1)
    def _():
        S = acc_ref[...]
        imp = S / (B * T)
        mean = jnp.mean(imp)
        var = jnp.mean((imp - mean) ** 2)
        val = var / (mean + 1e-9) ** 2 + jnp.sum(S) / (B * T * N_EXPERTS)
        aux_ref[...] = jnp.broadcast_to(val, (1, 1))


def _moe_fsmn(x, embed, lp, skip):
    in_d = x.shape[-1]
    out, _, aux = pl.pallas_call(
        functools.partial(_moe_kernel, skip=skip),
        grid=(B,),
        in_specs=[
            pl.BlockSpec((1, T, in_d), lambda b: (b, 0, 0)),
            pl.BlockSpec((1, T, D_MODEL), lambda b: (b, 0, 0)),
            pl.BlockSpec((D_MODEL, N_EXPERTS), lambda b: (0, 0)),
            pl.BlockSpec((N_EXPERTS, in_d, HIDDEN), lambda b: (0, 0, 0)),
            pl.BlockSpec((N_EXPERTS, HIDDEN), lambda b: (0, 0)),
            pl.BlockSpec((N_EXPERTS, HIDDEN, D_MODEL), lambda b: (0, 0, 0)),
            pl.BlockSpec((N_EXPERTS, D_MODEL), lambda b: (0, 0)),
            pl.BlockSpec((LB, D_MODEL), lambda b: (0, 0)),
            pl.BlockSpec((LA, D_MODEL), lambda b: (0, 0)),
        ],
        out_specs=[
            pl.BlockSpec((1, T, D_MODEL), lambda b: (b, 0, 0)),
            pl.BlockSpec((1, N_EXPERTS), lambda b: (0, 0)),
            pl.BlockSpec((1, 1), lambda b: (0, 0)),
        ],
        out_shape=[
            jax.ShapeDtypeStruct((B, T, D_MODEL), _F32),
            jax.ShapeDtypeStruct((1, N_EXPERTS), _F32),
            jax.ShapeDtypeStruct((1, 1), _F32),
        ],
    )(x, embed, lp["Wr"], lp["P"], lp["bp"], lp["V"], lp["bv"], lp["A"],
      lp["C"])
    return out, aux[0, 0]


def _proj_kernel(x_ref, W_ref, b_ref, o_ref):
    o_ref[0] = _dot(x_ref[0], W_ref[...]) + b_ref[...]


def _proj(x, W, bo):
    return pl.pallas_call(
        _proj_kernel,
        grid=(B,),
        in_specs=[
            pl.BlockSpec((1, T, D_MODEL), lambda b: (b, 0, 0)),
            pl.BlockSpec((D_MODEL, OUT_DIM), lambda b: (0, 0)),
            pl.BlockSpec((1, OUT_DIM), lambda b: (0, 0)),
        ],
        out_specs=pl.BlockSpec((1, T, OUT_DIM), lambda b: (b, 0, 0)),
        out_shape=jax.ShapeDtypeStruct((B, T, OUT_DIM), _F32),
    )(x, W, bo.reshape(1, -1))


def kernel(inputs, seq_len, params):
    mask = jnp.arange(T)[None, :] < seq_len[:, None]
    kmask = jnp.concatenate([mask, jnp.ones((B, N_MEM), bool)], axis=1)
    maskb = jnp.where(kmask, 0.0, -1e9).astype(_F32).reshape(B, 1, T + N_MEM)
    pe = jnp.asarray(_PE)

    xe = inputs
    for i, lp in enumerate(params["embed_fsmn"]):
        xe = _fsmn_plain(xe, lp, skip=(i > 0))
    embed = _san(xe, maskb, params["embed_san"])

    x = inputs
    aux = jnp.float32(0.0)
    for b_i, bp in enumerate(params["blocks"]):
        for i, lp in enumerate(bp["fsmn"]):
            skip = not (b_i == 0 and i == 0)
            x, a = _moe_fsmn(x, embed, lp, skip)
            aux = aux + a
        x = _san(x, maskb, bp["san"], pe=pe if b_i == 0 else None)

    out = _proj(x, params["Wout"], params["bout"])
    return out, aux
